# baseline (speedup 1.0000x reference)
import jax
import jax.numpy as jnp
from jax.experimental import pallas as pl


def _lrelu(v):
    return jnp.where(v >= 0, v, 0.01 * v)


def _graph_norm(x, gamma, beta, alpha, eps=1e-5):
    mean = jnp.mean(x, axis=0, keepdims=True)
    xc = x - alpha * mean
    var = jnp.mean(xc * xc, axis=0, keepdims=True)
    return gamma * xc / jnp.sqrt(var + eps) + beta


def _mm_kernel(x_ref, w_ref, b_ref, o_ref):
    o_ref[...] = jnp.dot(x_ref[...], w_ref[...],
                         preferred_element_type=jnp.float32) + b_ref[...]


def _pallas_mm(x, w, b):
    M, K = x.shape
    K2, Nc = w.shape
    return pl.pallas_call(
        _mm_kernel,
        out_shape=jax.ShapeDtypeStruct((M, Nc), jnp.float32),
        grid=(M // 2000,),
        in_specs=[
            pl.BlockSpec((2000, K), lambda i: (i, 0)),
            pl.BlockSpec((K, Nc), lambda i: (0, 0)),
            pl.BlockSpec((Nc,), lambda i: (0,)),
        ],
        out_specs=pl.BlockSpec((2000, Nc), lambda i: (i, 0)),
    )(x, w, b)


def kernel(x, x_struct, x_e, edge_index, W1e, b1e, W2e, b2e, W1n, b1n, W2n, b2n, Wq, bq, Wk, bk, att, g1, be1, a1, Wf, bf, g2, be2, a2, Wc1, bc1, Wc2, bc2):
    N = x.shape[0]
    M = x_e.shape[0]
    E = edge_index.shape[1]
    src = edge_index[0]
    dst = edge_index[1]
    xe = _lrelu(x_e @ W1e + b1e)
    xe = _lrelu(xe @ W2e + b2e)
    h = _lrelu(x @ W1n + b1n)
    ones = jnp.ones((E,), jnp.float32)
    deg_e = jnp.maximum(jax.ops.segment_sum(ones, dst, num_segments=M), 1.0)
    e_agg = jax.ops.segment_sum(h[src], dst, num_segments=M) / deg_e[:, None]
    deg_n = jnp.maximum(jax.ops.segment_sum(ones, src, num_segments=N), 1.0)
    n_agg = jax.ops.segment_sum(e_agg[dst], src, num_segments=N) / deg_n[:, None]
    h = _lrelu((h + n_agg) @ W2n + b2n)
    q = h @ Wq + bq
    k = xe @ Wk + bk
    H = q.shape[1]
    sq = q @ att[:H]
    sk = k @ att[H:]
    scores = _lrelu(sq[src] + sk[dst])
    smax = jax.ops.segment_max(scores, src, num_segments=N)
    ex = jnp.exp(scores - smax[src])
    denom = jax.ops.segment_sum(ex, src, num_segments=N)
    alpha_w = ex / (denom[src] + 1e-16)
    hn = jax.ops.segment_sum(alpha_w[:, None] * k[dst], src, num_segments=N)
    hm = jax.ops.segment_min(hn[src], dst, num_segments=M)
    hm = jnp.where(jnp.isfinite(hm), hm, 0.0)
    z = jnp.concatenate([hm, xe], axis=1)
    z = _graph_norm(z, g1, be1, a1)
    z = _lrelu(z @ Wf + bf)
    z = _graph_norm(z, g2, be2, a2)
    z = _lrelu(z)
    z = _lrelu(_pallas_mm(z, Wc1, bc1))
    z = _pallas_mm(z, Wc2, bc2)
    return z


# speedup vs baseline: 1.1437x; 1.1437x over previous
import functools

import jax
import jax.numpy as jnp
from jax import lax
from jax.experimental import pallas as pl
from jax.experimental.pallas import tpu as pltpu
from jax.experimental.pallas import tpu_sc as plsc

N = 10000
M = 10000
E = 320000
H = 128

_NC = 2
_NS = 16
_NW = _NC * _NS
_CHUNK = 125
_ROWS_PER_W = E // _CHUNK // _NW
_IDX_BLOCK = 40


def _lrelu(v):
    return jnp.where(v >= 0, v, 0.01 * v)


def _graph_norm(x, gamma, beta, alpha, eps=1e-5):
    mean = jnp.mean(x, axis=0, keepdims=True)
    xc = x - alpha * mean
    var = jnp.mean(xc * xc, axis=0, keepdims=True)
    return gamma * xc / jnp.sqrt(var + eps) + beta


def _seg_sum_body(table, gidx, sidx, zeros, out, gidx_v, sidx_v, rows0, rows1,
                  acc, gsem0, gsem1):
    c = lax.axis_index("c")
    s = lax.axis_index("s")
    w = s * _NC + c

    rows_per_sub = 624
    pltpu.sync_copy(zeros.at[pl.ds(s * rows_per_sub, rows_per_sub)],
                    acc.at[pl.ds(s * rows_per_sub, rows_per_sub)])

    @pl.when(s == 0)
    def _():
        pltpu.sync_copy(zeros.at[pl.ds(_NS * rows_per_sub, M - _NS * rows_per_sub)],
                        acc.at[pl.ds(_NS * rows_per_sub, M - _NS * rows_per_sub)])
    plsc.subcore_barrier()

    rows = (rows0, rows1)
    sems = (gsem0, gsem1)
    descs = [None, None]

    def start(j, b):
        descs[b] = pltpu.async_copy(table.at[gidx_v.at[j]], rows[b], sems[b])

    for blk in range(_ROWS_PER_W // _IDX_BLOCK):
        base = w * _ROWS_PER_W + blk * _IDX_BLOCK
        pltpu.sync_copy(gidx.at[pl.ds(base, _IDX_BLOCK)], gidx_v)
        pltpu.sync_copy(sidx.at[pl.ds(base, _IDX_BLOCK)], sidx_v)
        start(0, 0)
        for j in range(_IDX_BLOCK):
            b = j & 1
            if j + 1 < _IDX_BLOCK:
                start(j + 1, (j + 1) & 1)
            descs[b].wait()
            pltpu.sync_copy(rows[b], acc.at[sidx_v.at[j]], add=True)

    plsc.subcore_barrier()
    pltpu.sync_copy(acc.at[pl.ds(s * rows_per_sub, rows_per_sub)],
                    out.at[c].at[pl.ds(s * rows_per_sub, rows_per_sub)])

    @pl.when(s == 0)
    def _():
        pltpu.sync_copy(acc.at[pl.ds(_NS * rows_per_sub, M - _NS * rows_per_sub)],
                        out.at[c].at[pl.ds(_NS * rows_per_sub, M - _NS * rows_per_sub)])


@functools.partial(jax.jit, static_argnums=())
def _seg_sum(table, gidx_rows, sidx_rows, zeros):
    f = pl.kernel(
        _seg_sum_body,
        out_type=jax.ShapeDtypeStruct((_NC, M, H), jnp.float32),
        mesh=plsc.VectorSubcoreMesh(core_axis_name="c", subcore_axis_name="s"),
        scratch_types=[
            pltpu.VMEM((_IDX_BLOCK, _CHUNK), jnp.int32),
            pltpu.VMEM((_IDX_BLOCK, _CHUNK), jnp.int32),
            pltpu.VMEM((_CHUNK, H), jnp.float32),
            pltpu.VMEM((_CHUNK, H), jnp.float32),
            pltpu.VMEM_SHARED((M, H), jnp.float32),
            pltpu.SemaphoreType.DMA,
            pltpu.SemaphoreType.DMA,
        ],
    )
    return f(table, gidx_rows, sidx_rows, zeros)


def kernel(x, x_struct, x_e, edge_index, W1e, b1e, W2e, b2e, W1n, b1n, W2n, b2n, Wq, bq, Wk, bk, att, g1, be1, a1, Wf, bf, g2, be2, a2, Wc1, bc1, Wc2, bc2):
    src = edge_index[0]
    dst = edge_index[1]
    src_rows = src.reshape(E // _CHUNK, _CHUNK)
    dst_rows = dst.reshape(E // _CHUNK, _CHUNK)
    zeros_mh = jnp.zeros((M, H), jnp.float32)

    xe = _lrelu(x_e @ W1e + b1e)
    xe = _lrelu(xe @ W2e + b2e)
    h = _lrelu(x @ W1n + b1n)
    ones = jnp.ones((E,), jnp.float32)
    deg_e = jnp.maximum(jax.ops.segment_sum(ones, dst, num_segments=M), 1.0)
    p = _seg_sum(h, src_rows, dst_rows, zeros_mh)
    e_agg = (p[0] + p[1]) / deg_e[:, None]
    deg_n = jnp.maximum(jax.ops.segment_sum(ones, src, num_segments=N), 1.0)
    p = _seg_sum(e_agg, dst_rows, src_rows, zeros_mh)
    n_agg = (p[0] + p[1]) / deg_n[:, None]
    h = _lrelu((h + n_agg) @ W2n + b2n)
    q = h @ Wq + bq
    k = xe @ Wk + bk
    sq = q @ att[:H]
    sk = k @ att[H:]
    scores = _lrelu(sq[src] + sk[dst])
    smax = jax.ops.segment_max(scores, src, num_segments=N)
    ex = jnp.exp(scores - smax[src])
    denom = jax.ops.segment_sum(ex, src, num_segments=N)
    alpha_w = ex / (denom[src] + 1e-16)
    hn = jax.ops.segment_sum(alpha_w[:, None] * k[dst], src, num_segments=N)
    hm = jax.ops.segment_min(hn[src], dst, num_segments=M)
    hm = jnp.where(jnp.isfinite(hm), hm, 0.0)
    z = jnp.concatenate([hm, xe], axis=1)
    z = _graph_norm(z, g1, be1, a1)
    z = _lrelu(z @ Wf + bf)
    z = _graph_norm(z, g2, be2, a2)
    z = _lrelu(z)
    z = _lrelu(z @ Wc1 + bc1)
    z = z @ Wc2 + bc2
    return z


# speedup vs baseline: 5.4489x; 4.7643x over previous
import functools

import jax
import jax.numpy as jnp
from jax import lax
from jax.experimental import pallas as pl
from jax.experimental.pallas import tpu as pltpu
from jax.experimental.pallas import tpu_sc as plsc

N = 10000
M = 10000
E = 320000
H = 128

_NC = 2
_NS = 16
_NW = _NC * _NS
_CHUNK = 125
_ROWS_PER_W = E // _CHUNK // _NW
_IDX_BLOCK = 40


def _lrelu(v):
    return jnp.where(v >= 0, v, 0.01 * v)


def _graph_norm(x, gamma, beta, alpha, eps=1e-5):
    mean = jnp.mean(x, axis=0, keepdims=True)
    xc = x - alpha * mean
    var = jnp.mean(xc * xc, axis=0, keepdims=True)
    return gamma * xc / jnp.sqrt(var + eps) + beta


def _seg_sum_body(table, gidx, sidx, zeros, out, gidx_v, sidx_v, rows0, rows1,
                  acc, gsem0, gsem1):
    c = lax.axis_index("c")
    s = lax.axis_index("s")
    w = s * _NC + c

    rows_per_sub = 624
    pltpu.sync_copy(zeros.at[pl.ds(s * rows_per_sub, rows_per_sub)],
                    acc.at[pl.ds(s * rows_per_sub, rows_per_sub)])

    @pl.when(s == 0)
    def _():
        pltpu.sync_copy(zeros.at[pl.ds(_NS * rows_per_sub, M - _NS * rows_per_sub)],
                        acc.at[pl.ds(_NS * rows_per_sub, M - _NS * rows_per_sub)])
    plsc.subcore_barrier()

    rows = (rows0, rows1)
    sems = (gsem0, gsem1)
    descs = [None, None]

    def start(j, b):
        descs[b] = pltpu.async_copy(table.at[gidx_v.at[j]], rows[b], sems[b])

    for blk in range(_ROWS_PER_W // _IDX_BLOCK):
        base = w * _ROWS_PER_W + blk * _IDX_BLOCK
        pltpu.sync_copy(gidx.at[pl.ds(base, _IDX_BLOCK)], gidx_v)
        pltpu.sync_copy(sidx.at[pl.ds(base, _IDX_BLOCK)], sidx_v)
        start(0, 0)
        for j in range(_IDX_BLOCK):
            b = j & 1
            if j + 1 < _IDX_BLOCK:
                start(j + 1, (j + 1) & 1)
            descs[b].wait()
            pltpu.sync_copy(rows[b], acc.at[sidx_v.at[j]], add=True)

    plsc.subcore_barrier()
    pltpu.sync_copy(acc.at[pl.ds(s * rows_per_sub, rows_per_sub)],
                    out.at[c].at[pl.ds(s * rows_per_sub, rows_per_sub)])

    @pl.when(s == 0)
    def _():
        pltpu.sync_copy(acc.at[pl.ds(_NS * rows_per_sub, M - _NS * rows_per_sub)],
                        out.at[c].at[pl.ds(_NS * rows_per_sub, M - _NS * rows_per_sub)])


_EPW = E // _NW
_NV = _EPW // 16
_SLOTS = 8


def _attn_ex_body(sq, sk, srcf, dstf, bv, ex_out, denp_out,
                  sq_v, sk_v, src_v, dst_v, ex_v, b_v, dacc):
    c = lax.axis_index("c")
    s = lax.axis_index("s")
    w = s * _NC + c
    base = w * _EPW
    pltpu.sync_copy(sq, sq_v)
    pltpu.sync_copy(sk, sk_v)
    pltpu.sync_copy(srcf.at[pl.ds(base, _EPW)], src_v)
    pltpu.sync_copy(dstf.at[pl.ds(base, _EPW)], dst_v)
    pltpu.sync_copy(bv, b_v)
    b16 = b_v[...]
    iota = lax.iota(jnp.int32, 16)
    mlo = iota < _SLOTS
    mhi = jnp.logical_not(mlo)
    slot_off = (iota % _SLOTS) * N

    def zbody(v, _):
        dacc[pl.ds(v * 16, 16)] = jnp.zeros((16,), jnp.float32)
        return 0

    lax.fori_loop(0, _SLOTS * N // 16, zbody, 0)

    def body(v, _):
        s16 = src_v[pl.ds(v * 16, 16)]
        d16 = dst_v[pl.ds(v * 16, 16)]
        a = plsc.load_gather(sq_v, [s16])
        b = plsc.load_gather(sk_v, [d16])
        sc = a + b
        sc = jnp.where(sc >= 0, sc, 0.01 * sc)
        e = jnp.exp(sc - b16)
        ex_v[pl.ds(v * 16, 16)] = e
        didx = slot_off + s16
        plsc.addupdate_scatter(dacc, [didx], e, mask=mlo)
        plsc.addupdate_scatter(dacc, [didx], e, mask=mhi)
        return 0

    lax.fori_loop(0, _NV, body, 0)

    def rbody(v, _):
        acc = dacc[pl.ds(v * 16, 16)]
        for k in range(1, _SLOTS):
            acc = acc + dacc[pl.ds(k * N + v * 16, 16)]
        dacc[pl.ds(v * 16, 16)] = acc
        return 0

    lax.fori_loop(0, N // 16, rbody, 0)
    pltpu.sync_copy(ex_v, ex_out.at[pl.ds(base, _EPW)])
    pltpu.sync_copy(dacc.at[pl.ds(0, N)], denp_out.at[pl.ds(w * N, N)])


def _attn_ex(sq, sk, srcf, dstf, bv):
    f = pl.kernel(
        _attn_ex_body,
        compiler_params=pltpu.CompilerParams(needs_layout_passes=False),
        out_type=(jax.ShapeDtypeStruct((E,), jnp.float32),
                  jax.ShapeDtypeStruct((_NW * N,), jnp.float32)),
        mesh=plsc.VectorSubcoreMesh(core_axis_name="c", subcore_axis_name="s"),
        scratch_types=[
            pltpu.VMEM((N,), jnp.float32),
            pltpu.VMEM((M,), jnp.float32),
            pltpu.VMEM((_EPW,), jnp.int32),
            pltpu.VMEM((_EPW,), jnp.int32),
            pltpu.VMEM((_EPW,), jnp.float32),
            pltpu.VMEM((16,), jnp.float32),
            pltpu.VMEM((_SLOTS * N,), jnp.float32),
        ],
    )
    return f(sq, sk, srcf, dstf, bv)


_HN_BLK = 16


def _hn_body(ktab, src2d, dst2d, ex2d, rec, zeros, out,
             src_v, dst_v, ex_v, rec_v, al_v, rows0, rows1, acc, gsem0, gsem1):
    c = lax.axis_index("c")
    s = lax.axis_index("s")
    w = s * _NC + c

    rows_per_sub = 624
    pltpu.sync_copy(zeros.at[pl.ds(s * rows_per_sub, rows_per_sub)],
                    acc.at[pl.ds(s * rows_per_sub, rows_per_sub)])

    @pl.when(s == 0)
    def _():
        pltpu.sync_copy(zeros.at[pl.ds(_NS * rows_per_sub, N - _NS * rows_per_sub)],
                        acc.at[pl.ds(_NS * rows_per_sub, N - _NS * rows_per_sub)])

    pltpu.sync_copy(rec, rec_v)
    plsc.subcore_barrier()

    rows = (rows0, rows1)
    sems = (gsem0, gsem1)
    descs = [None, None]

    def start(j, b):
        descs[b] = pltpu.async_copy(ktab.at[dst_v.at[j]], rows[b], sems[b])

    offs = [v * 16 for v in range(7)] + [_CHUNK - 16]
    for blk in range(_ROWS_PER_W // _HN_BLK):
        rbase = w * _ROWS_PER_W + blk * _HN_BLK
        pltpu.sync_copy(src2d.at[pl.ds(rbase, _HN_BLK)], src_v)
        pltpu.sync_copy(dst2d.at[pl.ds(rbase, _HN_BLK)], dst_v)
        pltpu.sync_copy(ex2d.at[pl.ds(rbase, _HN_BLK)], ex_v)
        for j in range(_HN_BLK):
            for off in offs:
                s16 = src_v[j, pl.ds(off, 16)]
                e16 = ex_v[j, pl.ds(off, 16)]
                al_v[pl.ds(j * _CHUNK + off, 16)] = \
                    e16 * plsc.load_gather(rec_v, [s16])
        start(0, 0)
        for j in range(_HN_BLK):
            b = j & 1
            if j + 1 < _HN_BLK:
                start(j + 1, (j + 1) & 1)
            descs[b].wait()

            def ebody(i, _):
                sp = plsc.load_gather(al_v, [jnp.full((16,), j * _CHUNK, jnp.int32) + i])
                for t in range(8):
                    rows[b][i, pl.ds(t * 16, 16)] = rows[b][i, pl.ds(t * 16, 16)] * sp
                return 0

            lax.fori_loop(0, _CHUNK, ebody, 0)
            pltpu.sync_copy(rows[b], acc.at[src_v.at[j]], add=True)

    plsc.subcore_barrier()
    pltpu.sync_copy(acc.at[pl.ds(s * rows_per_sub, rows_per_sub)],
                    out.at[c].at[pl.ds(s * rows_per_sub, rows_per_sub)])

    @pl.when(s == 0)
    def _():
        pltpu.sync_copy(acc.at[pl.ds(_NS * rows_per_sub, N - _NS * rows_per_sub)],
                        out.at[c].at[pl.ds(_NS * rows_per_sub, N - _NS * rows_per_sub)])


def _hn_pass(ktab, src2d, dst2d, ex2d, rec, zeros):
    f = pl.kernel(
        _hn_body,
        compiler_params=pltpu.CompilerParams(needs_layout_passes=False),
        out_type=jax.ShapeDtypeStruct((_NC, N, H), jnp.float32),
        mesh=plsc.VectorSubcoreMesh(core_axis_name="c", subcore_axis_name="s"),
        scratch_types=[
            pltpu.VMEM((_HN_BLK, _CHUNK), jnp.int32),
            pltpu.VMEM((_HN_BLK, _CHUNK), jnp.int32),
            pltpu.VMEM((_HN_BLK, _CHUNK), jnp.float32),
            pltpu.VMEM((N,), jnp.float32),
            pltpu.VMEM((_HN_BLK * _CHUNK,), jnp.float32),
            pltpu.VMEM((_CHUNK, H), jnp.float32),
            pltpu.VMEM((_CHUNK, H), jnp.float32),
            pltpu.VMEM_SHARED((N, H), jnp.float32),
            pltpu.SemaphoreType.DMA,
            pltpu.SemaphoreType.DMA,
        ],
    )
    return f(ktab, src2d, dst2d, ex2d, rec, zeros)


@functools.partial(jax.jit, static_argnums=())
def _seg_sum(table, gidx_rows, sidx_rows, zeros):
    f = pl.kernel(
        _seg_sum_body,
        out_type=jax.ShapeDtypeStruct((_NC, M, H), jnp.float32),
        mesh=plsc.VectorSubcoreMesh(core_axis_name="c", subcore_axis_name="s"),
        scratch_types=[
            pltpu.VMEM((_IDX_BLOCK, _CHUNK), jnp.int32),
            pltpu.VMEM((_IDX_BLOCK, _CHUNK), jnp.int32),
            pltpu.VMEM((_CHUNK, H), jnp.float32),
            pltpu.VMEM((_CHUNK, H), jnp.float32),
            pltpu.VMEM_SHARED((M, H), jnp.float32),
            pltpu.SemaphoreType.DMA,
            pltpu.SemaphoreType.DMA,
        ],
    )
    return f(table, gidx_rows, sidx_rows, zeros)


def kernel(x, x_struct, x_e, edge_index, W1e, b1e, W2e, b2e, W1n, b1n, W2n, b2n, Wq, bq, Wk, bk, att, g1, be1, a1, Wf, bf, g2, be2, a2, Wc1, bc1, Wc2, bc2):
    src = edge_index[0]
    dst = edge_index[1]
    src_rows = src.reshape(E // _CHUNK, _CHUNK)
    dst_rows = dst.reshape(E // _CHUNK, _CHUNK)
    zeros_mh = jnp.zeros((M, H), jnp.float32)

    xe = _lrelu(x_e @ W1e + b1e)
    xe = _lrelu(xe @ W2e + b2e)
    h = _lrelu(x @ W1n + b1n)
    ones = jnp.ones((E,), jnp.float32)
    deg_e = jnp.maximum(jax.ops.segment_sum(ones, dst, num_segments=M), 1.0)
    p = _seg_sum(h, src_rows, dst_rows, zeros_mh)
    e_agg = (p[0] + p[1]) / deg_e[:, None]
    deg_n = jnp.maximum(jax.ops.segment_sum(ones, src, num_segments=N), 1.0)
    p = _seg_sum(e_agg, dst_rows, src_rows, zeros_mh)
    n_agg = (p[0] + p[1]) / deg_n[:, None]
    h = _lrelu((h + n_agg) @ W2n + b2n)
    q = h @ Wq + bq
    k = xe @ Wk + bk
    sq = q @ att[:H]
    sk = k @ att[H:]
    bscal = _lrelu(jnp.max(sq) + jnp.max(sk))
    bv = jnp.broadcast_to(bscal, (16,))
    ex, denp = _attn_ex(sq, sk, src, dst, bv)
    denom = jnp.sum(denp.reshape(_NW, N), axis=0)
    rec = 1.0 / (denom + 1e-16)
    ex_rows = ex.reshape(E // _CHUNK, _CHUNK)
    p = _hn_pass(k, src_rows, dst_rows, ex_rows, rec, zeros_mh)
    hn = p[0] + p[1]
    hm = jax.ops.segment_min(hn[src], dst, num_segments=M)
    hm = jnp.where(jnp.isfinite(hm), hm, 0.0)
    z = jnp.concatenate([hm, xe], axis=1)
    z = _graph_norm(z, g1, be1, a1)
    z = _lrelu(z @ Wf + bf)
    z = _graph_norm(z, g2, be2, a2)
    z = _lrelu(z)
    z = _lrelu(z @ Wc1 + bc1)
    z = z @ Wc2 + bc2
    return z
